# baseline (speedup 1.0000x reference)
import functools

import jax
import jax.numpy as jnp
from jax import lax
from jax.experimental import pallas as pl
from jax.experimental.pallas import tpu as pltpu
from jax.experimental.pallas import tpu_sc as plsc

TOKENS = 2048
HIDDEN = 1024
E = 8
TOPK = 2
NGROUP = 4
EG = E // NGROUP
DFF = 512
SHARED_INTER = 1024
RSF = 2.5

TB = 256
TBG = 128
NPAIR = TOKENS * TOPK
P = NPAIR + E * TBG
NB = P // TBG
NBP = 64

NTILE = 16
NWORK = 32
TPW = TOKENS // NWORK


def _relu2(x):
    return jnp.square(jnp.maximum(x, 0.0))


def _spl(s):
    return jnp.broadcast_to(s, (16,))


def _gate_block(x_ref, gw_ref, bias_ref, eidx_ref, tw_ref, cnt_ref, xb_ref):
    x = x_ref[...]
    logits = jnp.dot(x, gw_ref[...].T, preferred_element_type=jnp.float32)
    scores = jax.nn.sigmoid(logits)
    swb = scores + bias_ref[...]

    gs = swb.reshape(TB, NGROUP, EG).sum(axis=-1)
    gidx = lax.broadcasted_iota(jnp.int32, (TB, NGROUP), 1)
    g1 = jnp.argmax(gs, axis=1)
    gs2 = jnp.where(gidx == g1[:, None], -jnp.inf, gs)
    g2 = jnp.argmax(gs2, axis=1)

    eidx = lax.broadcasted_iota(jnp.int32, (TB, E), 1)
    egrp = eidx // EG
    emask = (egrp == g1[:, None]) | (egrp == g2[:, None])
    masked = jnp.where(emask, swb, -jnp.inf)
    e1 = jnp.argmax(masked, axis=1)
    m2 = jnp.where(eidx == e1[:, None], -jnp.inf, masked)
    e2 = jnp.argmax(m2, axis=1)
    oh1 = (eidx == e1[:, None]).astype(jnp.float32)
    oh2 = (eidx == e2[:, None]).astype(jnp.float32)
    s1 = jnp.sum(oh1 * scores, axis=1)
    s2 = jnp.sum(oh2 * scores, axis=1)
    rn = RSF / (s1 + s2 + 1e-20)

    eidx_ref[...] = jnp.concatenate([e1[:, None], e2[:, None]], axis=1)
    tw_ref[...] = jnp.concatenate([(s1 * rn)[:, None], (s2 * rn)[:, None]], axis=1)

    c1 = jnp.sum(oh1.astype(jnp.int32), axis=0)
    c2 = jnp.sum(oh2.astype(jnp.int32), axis=0)
    z8 = jnp.zeros((1, E), jnp.int32)
    cnt_ref[...] = jnp.concatenate(
        [jnp.concatenate([c1[None, :], z8], axis=1)[None],
         jnp.concatenate([c2[None, :], z8], axis=1)[None]], axis=1)

    xb_ref[...] = x.astype(jnp.bfloat16)


def _gate(x, gate_weight, bias):
    return pl.pallas_call(
        _gate_block,
        grid=(TOKENS // TB,),
        in_specs=[
            pl.BlockSpec((TB, HIDDEN), lambda i: (i, 0)),
            pl.BlockSpec((E, HIDDEN), lambda i: (0, 0)),
            pl.BlockSpec((E,), lambda i: (0,)),
        ],
        out_specs=[
            pl.BlockSpec((TB, TOPK), lambda i: (i, 0)),
            pl.BlockSpec((TB, TOPK), lambda i: (i, 0)),
            pl.BlockSpec((1, TOPK, 16), lambda i: (i, 0, 0)),
            pl.BlockSpec((TB, HIDDEN), lambda i: (i, 0)),
        ],
        out_shape=[
            jax.ShapeDtypeStruct((TOKENS, TOPK), jnp.int32),
            jax.ShapeDtypeStruct((TOKENS, TOPK), jnp.float32),
            jax.ShapeDtypeStruct((TOKENS // TB, TOPK, 16), jnp.int32),
            jax.ShapeDtypeStruct((TOKENS, HIDDEN), jnp.bfloat16),
        ],
    )(x, gate_weight, bias)


def _shared_block(x_ref, sw1_ref, sw2_ref, out_ref):
    h = _relu2(jnp.dot(x_ref[...], sw1_ref[...], preferred_element_type=jnp.float32))
    out_ref[...] = jnp.dot(h, sw2_ref[...], preferred_element_type=jnp.float32)


def _shared_mlp(x, sw1, sw2):
    return pl.pallas_call(
        _shared_block,
        grid=(TOKENS // TB,),
        in_specs=[
            pl.BlockSpec((TB, HIDDEN), lambda i: (i, 0)),
            pl.BlockSpec((HIDDEN, SHARED_INTER), lambda i: (0, 0)),
            pl.BlockSpec((SHARED_INTER, HIDDEN), lambda i: (0, 0)),
        ],
        out_specs=pl.BlockSpec((TB, HIDDEN), lambda i: (i, 0)),
        out_shape=jax.ShapeDtypeStruct((TOKENS, HIDDEN), jnp.float32),
    )(x, sw1, sw2)


def _routing_body(eidx_hbm, cnts_hbm, pos_hbm, bexp_hbm, meta_hbm,
                  ev, posv, cv, bexpv, metav):
    sid = lax.axis_index("s")
    base = sid * (NPAIR // NTILE)
    lane = lax.broadcasted_iota(jnp.int32, (16,), 0)
    zi = jnp.zeros((16,), jnp.int32)

    pltpu.sync_copy(eidx_hbm.at[pl.ds(base, 256)], ev)
    pltpu.sync_copy(cnts_hbm, cv)

    tot = zi
    pre = zi
    for kp in range(TOPK):
        for ip in range(8):
            row = cv[ip, kp]
            wp = kp * 8 + ip
            tot = tot + row
            pre = pre + row * _spl((sid > wp).astype(jnp.int32))
    pt = (tot + (TBG - 1)) // TBG * TBG
    offs = plsc.cumsum(pt) - pt
    start_vec = offs + pre

    for i in range(16):
        ids = ev[pl.ds(i * 16, 16)]
        pos_v = zi
        for e in range(E):
            m = ids == e
            mi = m.astype(jnp.int32)
            c = plsc.cumsum(mi)
            s_e = jnp.sum(jnp.where(lane == e, start_vec, zi))
            pos_v = jnp.where(m, _spl(s_e) + c - jnp.ones((16,), jnp.int32), pos_v)
            start_vec = start_vec + jnp.where(lane == e, _spl(jnp.sum(mi)), zi)
        posv[pl.ds(i * 16, 16)] = pos_v
    pltpu.sync_copy(posv, pos_hbm.at[pl.ds(base, 256)])

    @pl.when(sid == 0)
    def _():
        block_start = offs // TBG
        for j in range(NBP // 16):
            bvec = lane + j * 16
            acc = jnp.full((16,), -1, jnp.int32)
            for e in range(E):
                bs_e = jnp.sum(jnp.where(lane == e, block_start, zi))
                acc = acc + (bvec >= _spl(bs_e)).astype(jnp.int32)
            bexpv[pl.ds(j * 16, 16)] = jnp.minimum(
                jnp.maximum(acc, zi), jnp.full((16,), E - 1, jnp.int32))
        metav[...] = jnp.where(lane == 0, _spl(jnp.sum(pt) // TBG), zi)
        pltpu.sync_copy(bexpv, bexp_hbm)
        pltpu.sync_copy(metav, meta_hbm)


def _routing(eidx_flat, cnts):
    mesh = plsc.VectorSubcoreMesh(
        core_axis_name="c", subcore_axis_name="s", num_cores=1)
    f = pl.kernel(
        _routing_body,
        out_type=[
            jax.ShapeDtypeStruct((NPAIR,), jnp.int32),
            jax.ShapeDtypeStruct((NBP,), jnp.int32),
            jax.ShapeDtypeStruct((16,), jnp.int32),
        ],
        mesh=mesh,
        scratch_types=[
            pltpu.VMEM((256,), jnp.int32),
            pltpu.VMEM((256,), jnp.int32),
            pltpu.VMEM((8, TOPK, 16), jnp.int32),
            pltpu.VMEM((NBP,), jnp.int32),
            pltpu.VMEM((16,), jnp.int32),
        ],
        compiler_params=pltpu.CompilerParams(needs_layout_passes=False),
    )
    return f(eidx_flat, cnts)


HID2 = HIDDEN // 2


def _b2i(a):
    return lax.bitcast_convert_type(a.reshape(a.shape[0], HID2, 2), jnp.int32)


def _i2b(a):
    return lax.bitcast_convert_type(a, jnp.bfloat16).reshape(a.shape[0], HIDDEN)


def _scatter_body(x_hbm, pos_hbm, xs_hbm, xb, i0, i1, sem0, sem1):
    wid = lax.axis_index("s") * 2 + lax.axis_index("c")
    tbase = wid * TPW
    pltpu.sync_copy(x_hbm.at[pl.ds(tbase, TPW)], xb)
    pltpu.sync_copy(pos_hbm.at[pl.ds(tbase, TPW)], i0)
    pltpu.sync_copy(pos_hbm.at[pl.ds(TOKENS + tbase, TPW)], i1)
    c0 = pltpu.async_copy(xb, xs_hbm.at[i0], sem0)
    c1 = pltpu.async_copy(xb, xs_hbm.at[i1], sem1)
    c0.wait()
    c1.wait()


def _scatter(xbf, pos):
    mesh = plsc.VectorSubcoreMesh(
        core_axis_name="c", subcore_axis_name="s", num_cores=2)
    f = pl.kernel(
        _scatter_body,
        out_type=jax.ShapeDtypeStruct((P, HID2), jnp.int32),
        mesh=mesh,
        scratch_types=[
            pltpu.VMEM((TPW, HID2), jnp.int32),
            pltpu.VMEM((TPW,), jnp.int32),
            pltpu.VMEM((TPW,), jnp.int32),
            pltpu.SemaphoreType.DMA,
            pltpu.SemaphoreType.DMA,
        ],
        compiler_params=pltpu.CompilerParams(needs_layout_passes=False),
    )
    return f(xbf, pos)


def _gemm_block(bexp_ref, meta_ref, xs_ref, w1_ref, w2_ref, out_ref):
    b = pl.program_id(0)

    @pl.when(b < meta_ref[0])
    def _():
        h = _relu2(jnp.dot(xs_ref[...], w1_ref[0],
                           preferred_element_type=jnp.float32))
        out_ref[...] = jnp.dot(h.astype(jnp.bfloat16), w2_ref[0],
                               preferred_element_type=jnp.float32).astype(jnp.bfloat16)


def _gemm(bexp, meta, xs, w1b, w2b):
    grid_spec = pltpu.PrefetchScalarGridSpec(
        num_scalar_prefetch=2,
        grid=(NB,),
        in_specs=[
            pl.BlockSpec((TBG, HIDDEN), lambda b, bexp, meta: (b, 0)),
            pl.BlockSpec((1, HIDDEN, DFF), lambda b, bexp, meta: (bexp[b], 0, 0)),
            pl.BlockSpec((1, DFF, HIDDEN), lambda b, bexp, meta: (bexp[b], 0, 0)),
        ],
        out_specs=pl.BlockSpec((TBG, HIDDEN), lambda b, bexp, meta: (b, 0)),
    )
    return pl.pallas_call(
        _gemm_block,
        grid_spec=grid_spec,
        out_shape=jax.ShapeDtypeStruct((P, HIDDEN), jnp.bfloat16),
    )(bexp, meta, xs, w1b, w2b)


def _gather_body(ys_hbm, pos_hbm, y0_hbm, y1_hbm, i0, i1, b0, b1, sem0, sem1):
    wid = lax.axis_index("s") * 2 + lax.axis_index("c")
    tbase = wid * TPW
    pltpu.sync_copy(pos_hbm.at[pl.ds(tbase, TPW)], i0)
    pltpu.sync_copy(pos_hbm.at[pl.ds(TOKENS + tbase, TPW)], i1)
    c0 = pltpu.async_copy(ys_hbm.at[i0], b0, sem0)
    c1 = pltpu.async_copy(ys_hbm.at[i1], b1, sem1)
    c0.wait()
    c1.wait()
    pltpu.sync_copy(b0, y0_hbm.at[pl.ds(tbase, TPW)])
    pltpu.sync_copy(b1, y1_hbm.at[pl.ds(tbase, TPW)])


def _gather(ys, pos):
    mesh = plsc.VectorSubcoreMesh(
        core_axis_name="c", subcore_axis_name="s", num_cores=2)
    f = pl.kernel(
        _gather_body,
        out_type=[
            jax.ShapeDtypeStruct((TOKENS, HID2), jnp.int32),
            jax.ShapeDtypeStruct((TOKENS, HID2), jnp.int32),
        ],
        mesh=mesh,
        scratch_types=[
            pltpu.VMEM((TPW,), jnp.int32),
            pltpu.VMEM((TPW,), jnp.int32),
            pltpu.VMEM((TPW, HID2), jnp.int32),
            pltpu.VMEM((TPW, HID2), jnp.int32),
            pltpu.SemaphoreType.DMA,
            pltpu.SemaphoreType.DMA,
        ],
        compiler_params=pltpu.CompilerParams(needs_layout_passes=False),
    )
    return f(ys, pos)


def _combine_block(y0_ref, y1_ref, sh_ref, tw_ref, out_ref):
    tw = tw_ref[...]
    out_ref[...] = (sh_ref[...]
                    + tw[:, 0:1] * y0_ref[...].astype(jnp.float32)
                    + tw[:, 1:2] * y1_ref[...].astype(jnp.float32))


def _combine(y0, y1, sh, tw):
    return pl.pallas_call(
        _combine_block,
        grid=(TOKENS // TB,),
        in_specs=[
            pl.BlockSpec((TB, HIDDEN), lambda i: (i, 0)),
            pl.BlockSpec((TB, HIDDEN), lambda i: (i, 0)),
            pl.BlockSpec((TB, HIDDEN), lambda i: (i, 0)),
            pl.BlockSpec((TB, TOPK), lambda i: (i, 0)),
        ],
        out_specs=pl.BlockSpec((TB, HIDDEN), lambda i: (i, 0)),
        out_shape=jax.ShapeDtypeStruct((TOKENS, HIDDEN), jnp.float32),
    )(y0, y1, sh, tw)


def kernel(hidden_states, gate_weight, e_score_correction_bias, w1, w2, shared_w1, shared_w2):
    orig_shape = hidden_states.shape
    x = hidden_states.reshape(-1, HIDDEN)
    w1b = w1.astype(jnp.bfloat16)
    w2b = w2.astype(jnp.bfloat16)

    eidx, tw, cnts, xbf = _gate(x, gate_weight, e_score_correction_bias)
    eflat = eidx.T.reshape(NPAIR)

    pos, bexp, meta = _routing(eflat, cnts)
    xs = _i2b(_scatter(_b2i(xbf), pos))
    sh = _shared_mlp(x, shared_w1, shared_w2)
    ys = _gemm(bexp, meta, xs, w1b, w2b)
    y0, y1 = _gather(_b2i(ys), pos)
    out = _combine(_i2b(y0), _i2b(y1), sh, tw)
    return out.reshape(orig_shape)


# speedup vs baseline: 5.0389x; 5.0389x over previous
import jax
import jax.numpy as jnp
from jax import lax
from jax.experimental import pallas as pl
from jax.experimental.pallas import tpu as pltpu

TOKENS = 2048
HIDDEN = 1024
E = 8
TOPK = 2
NGROUP = 4
EG = E // NGROUP
DFF = 512
SHARED_INTER = 1024
RSF = 2.5

TB = 512


def _relu2(x):
    return jnp.square(jnp.maximum(x, 0.0))


def _gate_shared_block(x_ref, gw_ref, bias_ref, sw1_ref, sw2_ref,
                       gates_ref, sh_ref):
    x = x_ref[...]

    logits = jnp.dot(x, gw_ref[...].T, preferred_element_type=jnp.float32)
    scores = jax.nn.sigmoid(logits)
    swb = scores + bias_ref[...]

    gs = swb.reshape(TB, NGROUP, EG).sum(axis=-1)
    gidx = lax.broadcasted_iota(jnp.int32, (TB, NGROUP), 1)
    g1 = jnp.argmax(gs, axis=1)
    gs2 = jnp.where(gidx == g1[:, None], -jnp.inf, gs)
    g2 = jnp.argmax(gs2, axis=1)

    eidx = lax.broadcasted_iota(jnp.int32, (TB, E), 1)
    egrp = eidx // EG
    emask = (egrp == g1[:, None]) | (egrp == g2[:, None])
    masked = jnp.where(emask, swb, -jnp.inf)
    e1 = jnp.argmax(masked, axis=1)
    m2 = jnp.where(eidx == e1[:, None], -jnp.inf, masked)
    e2 = jnp.argmax(m2, axis=1)
    oh1 = (eidx == e1[:, None]).astype(jnp.float32)
    oh2 = (eidx == e2[:, None]).astype(jnp.float32)
    s1 = jnp.sum(oh1 * scores, axis=1)
    s2 = jnp.sum(oh2 * scores, axis=1)
    rn = RSF / (s1 + s2 + 1e-20)
    gates_ref[...] = oh1 * (s1 * rn)[:, None] + oh2 * (s2 * rn)[:, None]

    xb = x.astype(jnp.bfloat16)
    h = _relu2(jnp.dot(xb, sw1_ref[...], preferred_element_type=jnp.float32))
    sh_ref[...] = jnp.dot(h.astype(jnp.bfloat16), sw2_ref[...],
                          preferred_element_type=jnp.float32)


def _gate_shared(x, gate_weight, bias, sw1b, sw2b):
    return pl.pallas_call(
        _gate_shared_block,
        grid=(TOKENS // TB,),
        in_specs=[
            pl.BlockSpec((TB, HIDDEN), lambda i: (i, 0)),
            pl.BlockSpec((E, HIDDEN), lambda i: (0, 0)),
            pl.BlockSpec((E,), lambda i: (0,)),
            pl.BlockSpec((HIDDEN, SHARED_INTER), lambda i: (0, 0)),
            pl.BlockSpec((SHARED_INTER, HIDDEN), lambda i: (0, 0)),
        ],
        out_specs=[
            pl.BlockSpec((TB, E), lambda i: (i, 0)),
            pl.BlockSpec((TB, HIDDEN), lambda i: (i, 0)),
        ],
        out_shape=[
            jax.ShapeDtypeStruct((TOKENS, E), jnp.float32),
            jax.ShapeDtypeStruct((TOKENS, HIDDEN), jnp.float32),
        ],
    )(x, gate_weight, bias, sw1b, sw2b)


def _routed_block(x_ref, gates_ref, sh_ref, w1_ref, w2_ref, out_ref):
    e = pl.program_id(1)
    xb = x_ref[...].astype(jnp.bfloat16)
    h = _relu2(jnp.dot(xb, w1_ref[0], preferred_element_type=jnp.float32))
    y = jnp.dot(h.astype(jnp.bfloat16), w2_ref[0],
                preferred_element_type=jnp.float32)
    lane = lax.broadcasted_iota(jnp.int32, (TB, E), 1)
    g = jnp.sum(jnp.where(lane == e, gates_ref[...], 0.0), axis=1, keepdims=True)
    contrib = g * y

    @pl.when(e == 0)
    def _():
        out_ref[...] = sh_ref[...] + contrib

    @pl.when(e != 0)
    def _():
        out_ref[...] = out_ref[...] + contrib


def _routed(x, gates, sh, w1b, w2b):
    return pl.pallas_call(
        _routed_block,
        grid=(TOKENS // TB, E),
        in_specs=[
            pl.BlockSpec((TB, HIDDEN), lambda i, e: (i, 0)),
            pl.BlockSpec((TB, E), lambda i, e: (i, 0)),
            pl.BlockSpec((TB, HIDDEN), lambda i, e: (i, 0)),
            pl.BlockSpec((1, HIDDEN, DFF), lambda i, e: (e, 0, 0)),
            pl.BlockSpec((1, DFF, HIDDEN), lambda i, e: (e, 0, 0)),
        ],
        out_specs=pl.BlockSpec((TB, HIDDEN), lambda i, e: (i, 0)),
        out_shape=jax.ShapeDtypeStruct((TOKENS, HIDDEN), jnp.float32),
        compiler_params=pltpu.CompilerParams(
            dimension_semantics=("parallel", "arbitrary")),
    )(x, gates, sh, w1b, w2b)


def kernel(hidden_states, gate_weight, e_score_correction_bias, w1, w2, shared_w1, shared_w2):
    orig_shape = hidden_states.shape
    x = hidden_states.reshape(-1, HIDDEN)
    w1b = w1.astype(jnp.bfloat16)
    w2b = w2.astype(jnp.bfloat16)
    sw1b = shared_w1.astype(jnp.bfloat16)
    sw2b = shared_w2.astype(jnp.bfloat16)

    gates, sh = _gate_shared(x, gate_weight, e_score_correction_bias, sw1b, sw2b)
    out = _routed(x, gates, sh, w1b, w2b)
    return out.reshape(orig_shape)


# speedup vs baseline: 6.3079x; 1.2518x over previous
import jax
import jax.numpy as jnp
from jax import lax
from jax.experimental import pallas as pl
from jax.experimental.pallas import tpu as pltpu

TOKENS = 2048
HIDDEN = 1024
E = 8
TOPK = 2
NGROUP = 4
EG = E // NGROUP
DFF = 512
SHARED_INTER = 1024
RSF = 2.5

TB = 512


def _relu2(x):
    return jnp.square(jnp.maximum(x, 0.0))


def _moe_block(x_ref, gw_ref, bias_ref, w1_ref, w2_ref, sw1_ref, sw2_ref, out_ref):
    x = x_ref[...]

    logits = jnp.dot(x, gw_ref[...].T, preferred_element_type=jnp.float32)
    scores = jax.nn.sigmoid(logits)
    swb = scores + bias_ref[...]

    gs = swb.reshape(TB, NGROUP, EG).sum(axis=-1)
    gidx = lax.broadcasted_iota(jnp.int32, (TB, NGROUP), 1)
    g1 = jnp.argmax(gs, axis=1)
    gs2 = jnp.where(gidx == g1[:, None], -jnp.inf, gs)
    g2 = jnp.argmax(gs2, axis=1)

    eidx = lax.broadcasted_iota(jnp.int32, (TB, E), 1)
    egrp = eidx // EG
    emask = (egrp == g1[:, None]) | (egrp == g2[:, None])
    masked = jnp.where(emask, swb, -jnp.inf)
    e1 = jnp.argmax(masked, axis=1)
    m2 = jnp.where(eidx == e1[:, None], -jnp.inf, masked)
    e2 = jnp.argmax(m2, axis=1)
    oh1 = (eidx == e1[:, None]).astype(jnp.float32)
    oh2 = (eidx == e2[:, None]).astype(jnp.float32)
    s1 = jnp.sum(oh1 * scores, axis=1)
    s2 = jnp.sum(oh2 * scores, axis=1)
    rn = RSF / (s1 + s2 + 1e-20)
    gates = oh1 * (s1 * rn)[:, None] + oh2 * (s2 * rn)[:, None]

    xb = x.astype(jnp.bfloat16)

    h = _relu2(jnp.dot(xb, sw1_ref[...], preferred_element_type=jnp.float32))
    acc = jnp.dot(h.astype(jnp.bfloat16), sw2_ref[...],
                  preferred_element_type=jnp.float32)

    for e in range(E):
        he = _relu2(jnp.dot(xb, w1_ref[e], preferred_element_type=jnp.float32))
        ye = jnp.dot(he.astype(jnp.bfloat16), w2_ref[e],
                     preferred_element_type=jnp.float32)
        acc = acc + gates[:, e:e + 1] * ye

    out_ref[...] = acc


def kernel(hidden_states, gate_weight, e_score_correction_bias, w1, w2, shared_w1, shared_w2):
    orig_shape = hidden_states.shape
    x = hidden_states.reshape(-1, HIDDEN)
    w1b = w1.astype(jnp.bfloat16)
    w2b = w2.astype(jnp.bfloat16)
    sw1b = shared_w1.astype(jnp.bfloat16)
    sw2b = shared_w2.astype(jnp.bfloat16)

    grid = (TOKENS // TB,)
    out = pl.pallas_call(
        _moe_block,
        grid=grid,
        in_specs=[
            pl.BlockSpec((TB, HIDDEN), lambda i: (i, 0)),
            pl.BlockSpec((E, HIDDEN), lambda i: (0, 0)),
            pl.BlockSpec((E,), lambda i: (0,)),
            pl.BlockSpec((E, HIDDEN, DFF), lambda i: (0, 0, 0)),
            pl.BlockSpec((E, DFF, HIDDEN), lambda i: (0, 0, 0)),
            pl.BlockSpec((HIDDEN, SHARED_INTER), lambda i: (0, 0)),
            pl.BlockSpec((SHARED_INTER, HIDDEN), lambda i: (0, 0)),
        ],
        out_specs=pl.BlockSpec((TB, HIDDEN), lambda i: (i, 0)),
        out_shape=jax.ShapeDtypeStruct((TOKENS, HIDDEN), jnp.float32),
    )(x, gate_weight, e_score_correction_bias, w1b, w2b, sw1b, sw2b)
    return out.reshape(orig_shape)
